# Initial kernel scaffold; baseline (speedup 1.0000x reference)
import jax
import jax.numpy as jnp
from jax.experimental import pallas as pl


def kernel(x, edge_index, e_h, norm, W_inv, b_inv, bias_v):
    raise NotImplementedError("write your pallas kernel here")


# speedup vs baseline: 4.4758x; 4.4758x over previous
import jax
import jax.numpy as jnp
from jax import lax
from jax.experimental import pallas as pl
from jax.experimental.pallas import tpu as pltpu
from jax.experimental.pallas import tpu_sc as plsc

N_NODES = 10000
N_EDGES = 160000
D = 256
HALF = 128
LANES = 16
NS = 16
EROWS = N_EDGES // HALF
EROWS_PAD = 1280
EROWS_PER_TILE = 80
IBLK = 16
R = 80
NCHUNKS = N_NODES // R


def _tile_ranges():
    s = lax.axis_index("s")
    row_start = s * EROWS_PER_TILE
    row_end = jnp.minimum(row_start + EROWS_PER_TILE, EROWS)
    nblocks = (row_end - row_start + (IBLK - 1)) // IBLK
    my_nchunks = jnp.where(s < NCHUNKS - 7 * NS, 8, 7)
    return s, row_start, row_end, nblocks, my_nchunks


def _fill_rows(buf, nrows, value):
    v16 = jnp.full((LANES,), value, jnp.float32)

    def frow(r, _):
        for j in range(HALF // LANES):
            buf[r, pl.ds(j * LANES, LANES)] = v16
        return 0
    lax.fori_loop(0, nrows, frow, 0)


def _zero_shared(s, my_nchunks, dbuf, acc_sh):
    _fill_rows(dbuf, R, 0.0)

    def zchunk(i, _):
        rb = (s + NS * i) * R
        pltpu.sync_copy(dbuf.at[pl.ds(0, R)], acc_sh.at[pl.ds(rb, R)])
        return 0
    lax.fori_loop(0, my_nchunks, zchunk, 0)


def _copy_out(s, my_nchunks, acc_sh, dbuf, out_hbm):
    def ochunk(i, _):
        rb = (s + NS * i) * R
        pltpu.sync_copy(acc_sh.at[pl.ds(rb, R)], dbuf.at[pl.ds(0, R)])
        pltpu.sync_copy(dbuf.at[pl.ds(0, R)], out_hbm.at[pl.ds(rb, R)])
        return 0
    lax.fori_loop(0, my_nchunks, ochunk, 0)


def _sc_a_body(ehlo_hbm, ehhi_hbm, dst_hbm, s0_hbm, s1_hbm,
               acc_sh, dsti, dbuf):
    c = lax.axis_index("c")
    s, row_start, row_end, nblocks, my_nchunks = _tile_ranges()

    _zero_shared(s, my_nchunks, dbuf, acc_sh)
    plsc.subcore_barrier()

    def scatter_half(eh_hbm):
        def s1_block(b, _):
            jb = row_start + b * IBLK
            pltpu.sync_copy(dst_hbm.at[pl.ds(jb, IBLK)], dsti)

            def s1(jj, _):
                j = jb + jj
                pltpu.sync_copy(eh_hbm.at[pl.ds(j * HALF, HALF)], dbuf)
                pltpu.sync_copy(dbuf, acc_sh.at[dsti.at[jj]], add=True)
                return 0
            lax.fori_loop(0, jnp.minimum(row_end - jb, IBLK), s1, 0)
            return 0
        lax.fori_loop(0, nblocks, s1_block, 0)

    @pl.when(c == 0)
    def _():
        scatter_half(ehlo_hbm)

    @pl.when(c == 1)
    def _():
        scatter_half(ehhi_hbm)

    plsc.subcore_barrier()

    @pl.when(c == 0)
    def _():
        _copy_out(s, my_nchunks, acc_sh, dbuf, s0_hbm)

    @pl.when(c == 1)
    def _():
        _copy_out(s, my_nchunks, acc_sh, dbuf, s1_hbm)


def _sc_c_body(dst_hbm, cnt_hbm, cnt_sh, dsti, dbuf):
    s, row_start, row_end, nblocks, my_nchunks = _tile_ranges()

    _zero_shared(s, my_nchunks, dbuf, cnt_sh)
    plsc.subcore_barrier()
    _fill_rows(dbuf, HALF, 1.0)

    def c_block(b, _):
        jb = row_start + b * IBLK
        pltpu.sync_copy(dst_hbm.at[pl.ds(jb, IBLK)], dsti)

        def cs(jj, _):
            pltpu.sync_copy(dbuf, cnt_sh.at[dsti.at[jj]], add=True)
            return 0
        lax.fori_loop(0, jnp.minimum(row_end - jb, IBLK), cs, 0)
        return 0
    lax.fori_loop(0, nblocks, c_block, 0)
    plsc.subcore_barrier()

    _copy_out(s, my_nchunks, cnt_sh, dbuf, cnt_hbm)


def _sc_b_body(h0_hbm, h1_hbm, src_hbm, dst_hbm, o0_hbm, o1_hbm,
               acc_sh, dsti, srci, dbuf, sem):
    c = lax.axis_index("c")
    s, row_start, row_end, nblocks, my_nchunks = _tile_ranges()

    _zero_shared(s, my_nchunks, dbuf, acc_sh)
    plsc.subcore_barrier()

    def gather_half(hor_hbm):
        def s2_block(b, _):
            jb = row_start + b * IBLK
            pltpu.sync_copy(dst_hbm.at[pl.ds(jb, IBLK)], dsti)
            pltpu.sync_copy(src_hbm.at[pl.ds(jb, IBLK)], srci)

            def s2(jj, _):
                pltpu.async_copy(hor_hbm.at[srci.at[jj]], dbuf, sem).wait()
                pltpu.sync_copy(dbuf, acc_sh.at[dsti.at[jj]], add=True)
                return 0
            lax.fori_loop(0, jnp.minimum(row_end - jb, IBLK), s2, 0)
            return 0
        lax.fori_loop(0, nblocks, s2_block, 0)

    @pl.when(c == 0)
    def _():
        gather_half(h0_hbm)

    @pl.when(c == 1)
    def _():
        gather_half(h1_hbm)

    plsc.subcore_barrier()

    @pl.when(c == 0)
    def _():
        _copy_out(s, my_nchunks, acc_sh, dbuf, o0_hbm)

    @pl.when(c == 1)
    def _():
        _copy_out(s, my_nchunks, acc_sh, dbuf, o1_hbm)


def _tc_hor_body(x_ref, s0_ref, s1_ref, cnt_ref, h0_ref, h1_ref):
    inv = 1.0 / jnp.maximum(cnt_ref[:, 0:1], 1.0)
    h0_ref[...] = x_ref[:, :HALF] * s0_ref[...] * inv
    h1_ref[...] = x_ref[:, HALF:] * s1_ref[...] * inv


def _tc_out_body(o0_ref, o1_ref, cnt_ref, w0_ref, w1_ref, b_ref, norm_ref,
                 bias_ref, out_ref):
    acc = jnp.dot(o0_ref[...], w0_ref[...],
                  preferred_element_type=jnp.float32)
    acc += jnp.dot(o1_ref[...], w1_ref[...],
                   preferred_element_type=jnp.float32)
    acc += cnt_ref[:, 0:1] * b_ref[...]
    out_ref[...] = acc * norm_ref[...] + bias_ref[...]


def kernel(x, edge_index, e_h, norm, W_inv, b_inv, bias_v):
    ei = edge_index.astype(jnp.int32)
    pad = EROWS_PAD * HALF - N_EDGES
    src = jnp.pad(ei[0], (0, pad)).reshape(EROWS_PAD, HALF)
    dst = jnp.pad(ei[1], (0, pad)).reshape(EROWS_PAD, HALF)
    eh_lo = jnp.asarray(e_h[:, :HALF])
    eh_hi = jnp.asarray(e_h[:, HALF:])

    mesh = plsc.VectorSubcoreMesh(core_axis_name="c", subcore_axis_name="s")
    f32 = jnp.float32

    sc_a = pl.kernel(
        _sc_a_body,
        out_type=[
            jax.ShapeDtypeStruct((N_NODES, HALF), f32),
            jax.ShapeDtypeStruct((N_NODES, HALF), f32),
        ],
        mesh=mesh,
        scratch_types=[
            pltpu.VMEM_SHARED((N_NODES, HALF), f32),
            pltpu.VMEM((IBLK, HALF), jnp.int32),
            pltpu.VMEM((HALF, HALF), f32),
        ],
    )
    s0, s1 = sc_a(eh_lo, eh_hi, dst)

    sc_c = pl.kernel(
        _sc_c_body,
        out_type=jax.ShapeDtypeStruct((N_NODES, HALF), f32),
        mesh=mesh,
        scratch_types=[
            pltpu.VMEM_SHARED((N_NODES, HALF), f32),
            pltpu.VMEM((IBLK, HALF), jnp.int32),
            pltpu.VMEM((HALF, HALF), f32),
        ],
    )
    cnt = sc_c(dst)

    blk = 2000
    grid = N_NODES // blk
    h0, h1 = pl.pallas_call(
        _tc_hor_body,
        grid=(grid,),
        in_specs=[
            pl.BlockSpec((blk, D), lambda i: (i, 0)),
            pl.BlockSpec((blk, HALF), lambda i: (i, 0)),
            pl.BlockSpec((blk, HALF), lambda i: (i, 0)),
            pl.BlockSpec((blk, HALF), lambda i: (i, 0)),
        ],
        out_specs=[
            pl.BlockSpec((blk, HALF), lambda i: (i, 0)),
            pl.BlockSpec((blk, HALF), lambda i: (i, 0)),
        ],
        out_shape=[
            jax.ShapeDtypeStruct((N_NODES, HALF), f32),
            jax.ShapeDtypeStruct((N_NODES, HALF), f32),
        ],
    )(x, s0, s1, cnt)

    sc_b = pl.kernel(
        _sc_b_body,
        out_type=[
            jax.ShapeDtypeStruct((N_NODES, HALF), f32),
            jax.ShapeDtypeStruct((N_NODES, HALF), f32),
        ],
        mesh=mesh,
        scratch_types=[
            pltpu.VMEM_SHARED((N_NODES, HALF), f32),
            pltpu.VMEM((IBLK, HALF), jnp.int32),
            pltpu.VMEM((IBLK, HALF), jnp.int32),
            pltpu.VMEM((HALF, HALF), f32),
            pltpu.SemaphoreType.DMA,
        ],
    )
    o0, o1 = sc_b(h0, h1, src, dst)

    out = pl.pallas_call(
        _tc_out_body,
        grid=(grid,),
        in_specs=[
            pl.BlockSpec((blk, HALF), lambda i: (i, 0)),
            pl.BlockSpec((blk, HALF), lambda i: (i, 0)),
            pl.BlockSpec((blk, HALF), lambda i: (i, 0)),
            pl.BlockSpec((HALF, D), lambda i: (0, 0)),
            pl.BlockSpec((HALF, D), lambda i: (0, 0)),
            pl.BlockSpec((1, D), lambda i: (0, 0)),
            pl.BlockSpec((blk, 1), lambda i: (i, 0)),
            pl.BlockSpec((1, D), lambda i: (0, 0)),
        ],
        out_specs=pl.BlockSpec((blk, D), lambda i: (i, 0)),
        out_shape=jax.ShapeDtypeStruct((N_NODES, D), f32),
    )(o0, o1, cnt, W_inv[:HALF], W_inv[HALF:], b_inv.reshape(1, D),
      norm, bias_v.reshape(1, D))
    return out


# speedup vs baseline: 5.4209x; 1.2112x over previous
import jax
import jax.numpy as jnp
from jax import lax
from jax.experimental import pallas as pl
from jax.experimental.pallas import tpu as pltpu
from jax.experimental.pallas import tpu_sc as plsc

N_NODES = 10000
N_EDGES = 160000
D = 256
HALF = 128
LANES = 16
NS = 16
EROWS = N_EDGES // HALF
EROWS_PAD = 1280
EROWS_PER_TILE = 80
IBLK = 16
R = 80
NCHUNKS = N_NODES // R


def _tile_ranges():
    s = lax.axis_index("s")
    row_start = s * EROWS_PER_TILE
    row_end = jnp.minimum(row_start + EROWS_PER_TILE, EROWS)
    nblocks = (row_end - row_start + (IBLK - 1)) // IBLK
    my_nchunks = jnp.where(s < NCHUNKS - 7 * NS, 8, 7)
    return s, row_start, row_end, nblocks, my_nchunks


def _fill_rows(buf, nrows, value):
    v16 = jnp.full((LANES,), value, jnp.float32)

    def frow(r, _):
        for j in range(HALF // LANES):
            buf[r, pl.ds(j * LANES, LANES)] = v16
        return 0
    lax.fori_loop(0, nrows, frow, 0)


def _zero_shared(s, my_nchunks, dbuf, acc_sh):
    _fill_rows(dbuf, R, 0.0)

    def zchunk(i, _):
        rb = (s + NS * i) * R
        pltpu.sync_copy(dbuf.at[pl.ds(0, R)], acc_sh.at[pl.ds(rb, R)])
        return 0
    lax.fori_loop(0, my_nchunks, zchunk, 0)


def _copy_out(s, my_nchunks, acc_sh, dbuf, out_hbm):
    def ochunk(i, _):
        rb = (s + NS * i) * R
        pltpu.sync_copy(acc_sh.at[pl.ds(rb, R)], dbuf.at[pl.ds(0, R)])
        pltpu.sync_copy(dbuf.at[pl.ds(0, R)], out_hbm.at[pl.ds(rb, R)])
        return 0
    lax.fori_loop(0, my_nchunks, ochunk, 0)


def _sc_a_body(eh_hbm, dst_hbm, s0_hbm, s1_hbm,
               acc_sh, dsti, dbuf):
    c = lax.axis_index("c")
    s, row_start, row_end, nblocks, my_nchunks = _tile_ranges()

    _zero_shared(s, my_nchunks, dbuf, acc_sh)
    plsc.subcore_barrier()

    def scatter_half(col0):
        def s1_block(b, _):
            jb = row_start + b * IBLK
            pltpu.sync_copy(dst_hbm.at[pl.ds(jb, IBLK)], dsti)

            def s1(jj, _):
                j = jb + jj
                pltpu.sync_copy(
                    eh_hbm.at[pl.ds(j * HALF, HALF), pl.ds(col0, HALF)],
                    dbuf)
                pltpu.sync_copy(dbuf, acc_sh.at[dsti.at[jj]], add=True)
                return 0
            lax.fori_loop(0, jnp.minimum(row_end - jb, IBLK), s1, 0)
            return 0
        lax.fori_loop(0, nblocks, s1_block, 0)

    @pl.when(c == 0)
    def _():
        scatter_half(0)

    @pl.when(c == 1)
    def _():
        scatter_half(HALF)

    plsc.subcore_barrier()

    @pl.when(c == 0)
    def _():
        _copy_out(s, my_nchunks, acc_sh, dbuf, s0_hbm)

    @pl.when(c == 1)
    def _():
        _copy_out(s, my_nchunks, acc_sh, dbuf, s1_hbm)


def _sc_c_body(dst_hbm, cnt_hbm, cnt_sh, dsti, dbuf):
    s, row_start, row_end, nblocks, my_nchunks = _tile_ranges()

    _zero_shared(s, my_nchunks, dbuf, cnt_sh)
    plsc.subcore_barrier()
    _fill_rows(dbuf, HALF, 1.0)

    def c_block(b, _):
        jb = row_start + b * IBLK
        pltpu.sync_copy(dst_hbm.at[pl.ds(jb, IBLK)], dsti)

        def cs(jj, _):
            pltpu.sync_copy(dbuf, cnt_sh.at[dsti.at[jj]], add=True)
            return 0
        lax.fori_loop(0, jnp.minimum(row_end - jb, IBLK), cs, 0)
        return 0
    lax.fori_loop(0, nblocks, c_block, 0)
    plsc.subcore_barrier()

    _copy_out(s, my_nchunks, cnt_sh, dbuf, cnt_hbm)


def _sc_b_body(h0_hbm, h1_hbm, src_hbm, dst_hbm, o0_hbm, o1_hbm,
               acc_sh, dsti, srci, dbuf, sem):
    c = lax.axis_index("c")
    s, row_start, row_end, nblocks, my_nchunks = _tile_ranges()

    _zero_shared(s, my_nchunks, dbuf, acc_sh)
    plsc.subcore_barrier()

    def gather_half(hor_hbm):
        def s2_block(b, _):
            jb = row_start + b * IBLK
            pltpu.sync_copy(dst_hbm.at[pl.ds(jb, IBLK)], dsti)
            pltpu.sync_copy(src_hbm.at[pl.ds(jb, IBLK)], srci)

            def s2(jj, _):
                pltpu.async_copy(hor_hbm.at[srci.at[jj]], dbuf, sem).wait()
                pltpu.sync_copy(dbuf, acc_sh.at[dsti.at[jj]], add=True)
                return 0
            lax.fori_loop(0, jnp.minimum(row_end - jb, IBLK), s2, 0)
            return 0
        lax.fori_loop(0, nblocks, s2_block, 0)

    @pl.when(c == 0)
    def _():
        gather_half(h0_hbm)

    @pl.when(c == 1)
    def _():
        gather_half(h1_hbm)

    plsc.subcore_barrier()

    @pl.when(c == 0)
    def _():
        _copy_out(s, my_nchunks, acc_sh, dbuf, o0_hbm)

    @pl.when(c == 1)
    def _():
        _copy_out(s, my_nchunks, acc_sh, dbuf, o1_hbm)


def _tc_hor_body(x_ref, s0_ref, s1_ref, cnt_ref, h0_ref, h1_ref):
    inv = 1.0 / jnp.maximum(cnt_ref[:, 0:1], 1.0)
    h0_ref[...] = x_ref[:, :HALF] * s0_ref[...] * inv
    h1_ref[...] = x_ref[:, HALF:] * s1_ref[...] * inv


def _tc_out_body(o0_ref, o1_ref, cnt_ref, w0_ref, w1_ref, b_ref, norm_ref,
                 bias_ref, out_ref):
    acc = jnp.dot(o0_ref[...], w0_ref[...],
                  preferred_element_type=jnp.float32)
    acc += jnp.dot(o1_ref[...], w1_ref[...],
                   preferred_element_type=jnp.float32)
    acc += cnt_ref[:, 0:1] * b_ref[...]
    out_ref[...] = acc * norm_ref[...] + bias_ref[...]


def kernel(x, edge_index, e_h, norm, W_inv, b_inv, bias_v):
    ei = edge_index.astype(jnp.int32)
    pad = EROWS_PAD * HALF - N_EDGES
    src = jnp.pad(ei[0], (0, pad)).reshape(EROWS_PAD, HALF)
    dst = jnp.pad(ei[1], (0, pad)).reshape(EROWS_PAD, HALF)
    mesh = plsc.VectorSubcoreMesh(core_axis_name="c", subcore_axis_name="s")
    f32 = jnp.float32

    sc_a = pl.kernel(
        _sc_a_body,
        out_type=[
            jax.ShapeDtypeStruct((N_NODES, HALF), f32),
            jax.ShapeDtypeStruct((N_NODES, HALF), f32),
        ],
        mesh=mesh,
        scratch_types=[
            pltpu.VMEM_SHARED((N_NODES, HALF), f32),
            pltpu.VMEM((IBLK, HALF), jnp.int32),
            pltpu.VMEM((HALF, HALF), f32),
        ],
    )
    s0, s1 = sc_a(e_h, dst)

    sc_c = pl.kernel(
        _sc_c_body,
        out_type=jax.ShapeDtypeStruct((N_NODES, HALF), f32),
        mesh=mesh,
        scratch_types=[
            pltpu.VMEM_SHARED((N_NODES, HALF), f32),
            pltpu.VMEM((IBLK, HALF), jnp.int32),
            pltpu.VMEM((HALF, HALF), f32),
        ],
    )
    cnt = sc_c(dst)

    blk = 2000
    grid = N_NODES // blk
    h0, h1 = pl.pallas_call(
        _tc_hor_body,
        grid=(grid,),
        in_specs=[
            pl.BlockSpec((blk, D), lambda i: (i, 0)),
            pl.BlockSpec((blk, HALF), lambda i: (i, 0)),
            pl.BlockSpec((blk, HALF), lambda i: (i, 0)),
            pl.BlockSpec((blk, HALF), lambda i: (i, 0)),
        ],
        out_specs=[
            pl.BlockSpec((blk, HALF), lambda i: (i, 0)),
            pl.BlockSpec((blk, HALF), lambda i: (i, 0)),
        ],
        out_shape=[
            jax.ShapeDtypeStruct((N_NODES, HALF), f32),
            jax.ShapeDtypeStruct((N_NODES, HALF), f32),
        ],
    )(x, s0, s1, cnt)

    sc_b = pl.kernel(
        _sc_b_body,
        out_type=[
            jax.ShapeDtypeStruct((N_NODES, HALF), f32),
            jax.ShapeDtypeStruct((N_NODES, HALF), f32),
        ],
        mesh=mesh,
        scratch_types=[
            pltpu.VMEM_SHARED((N_NODES, HALF), f32),
            pltpu.VMEM((IBLK, HALF), jnp.int32),
            pltpu.VMEM((IBLK, HALF), jnp.int32),
            pltpu.VMEM((HALF, HALF), f32),
            pltpu.SemaphoreType.DMA,
        ],
    )
    o0, o1 = sc_b(h0, h1, src, dst)

    out = pl.pallas_call(
        _tc_out_body,
        grid=(grid,),
        in_specs=[
            pl.BlockSpec((blk, HALF), lambda i: (i, 0)),
            pl.BlockSpec((blk, HALF), lambda i: (i, 0)),
            pl.BlockSpec((blk, HALF), lambda i: (i, 0)),
            pl.BlockSpec((HALF, D), lambda i: (0, 0)),
            pl.BlockSpec((HALF, D), lambda i: (0, 0)),
            pl.BlockSpec((1, D), lambda i: (0, 0)),
            pl.BlockSpec((blk, 1), lambda i: (i, 0)),
            pl.BlockSpec((1, D), lambda i: (0, 0)),
        ],
        out_specs=pl.BlockSpec((blk, D), lambda i: (i, 0)),
        out_shape=jax.ShapeDtypeStruct((N_NODES, D), f32),
    )(o0, o1, cnt, W_inv[:HALF], W_inv[HALF:], b_inv.reshape(1, D),
      norm, bias_v.reshape(1, D))
    return out


# speedup vs baseline: 6.6910x; 1.2343x over previous
import jax
import jax.numpy as jnp
from jax import lax
from jax.experimental import pallas as pl
from jax.experimental.pallas import tpu as pltpu
from jax.experimental.pallas import tpu_sc as plsc

N_NODES = 10000
N_EDGES = 160000
D = 256
HALF = 128
LANES = 16
NS = 16
EROWS = N_EDGES // HALF
EROWS_PAD = 1280
EROWS_PER_TILE = 80
IBLK = 16
R = 80
NCHUNKS = N_NODES // R


def _tile_ranges():
    s = lax.axis_index("s")
    row_start = s * EROWS_PER_TILE
    row_end = jnp.minimum(row_start + EROWS_PER_TILE, EROWS)
    nblocks = (row_end - row_start + (IBLK - 1)) // IBLK
    my_nchunks = jnp.where(s < NCHUNKS - 7 * NS, 8, 7)
    return s, row_start, row_end, nblocks, my_nchunks


def _fill_rows(buf, nrows, value):
    v16 = jnp.full((LANES,), value, jnp.float32)

    def frow(r, _):
        for j in range(HALF // LANES):
            buf[r, pl.ds(j * LANES, LANES)] = v16
        return 0
    lax.fori_loop(0, nrows, frow, 0)


def _zero_shared(s, my_nchunks, dbuf, acc_sh):
    _fill_rows(dbuf, R, 0.0)

    def zchunk(i, _):
        rb = (s + NS * i) * R
        pltpu.sync_copy(dbuf.at[pl.ds(0, R)], acc_sh.at[pl.ds(rb, R)])
        return 0
    lax.fori_loop(0, my_nchunks, zchunk, 0)


def _copy_out(s, my_nchunks, acc_sh, dbuf, out_hbm):
    def ochunk(i, _):
        rb = (s + NS * i) * R
        pltpu.sync_copy(acc_sh.at[pl.ds(rb, R)], dbuf.at[pl.ds(0, R)])
        pltpu.sync_copy(dbuf.at[pl.ds(0, R)], out_hbm.at[pl.ds(rb, R)])
        return 0
    lax.fori_loop(0, my_nchunks, ochunk, 0)


def _sc_a_body(eh_hbm, dst_hbm, s0_hbm, s1_hbm,
               acc_sh, dsti, dbuf, dbufb, sema, semb):
    c = lax.axis_index("c")
    s, row_start, row_end, nblocks, my_nchunks = _tile_ranges()

    _zero_shared(s, my_nchunks, dbuf, acc_sh)
    plsc.subcore_barrier()

    def scatter_half(col0):
        def eh_row(j):
            return eh_hbm.at[pl.ds(j * HALF, HALF), pl.ds(col0, HALF)]

        def s1_block(b, _):
            jb = row_start + b * IBLK
            pltpu.sync_copy(dst_hbm.at[pl.ds(jb, IBLK)], dsti)
            npairs = jnp.minimum(row_end - jb, IBLK) // 2
            pltpu.async_copy(eh_row(jb), dbuf, sema)

            def pair(p, _):
                j = jb + 2 * p
                pltpu.make_async_copy(eh_row(j), dbuf, sema).wait()
                pltpu.async_copy(eh_row(j + 1), dbufb, semb)
                pltpu.sync_copy(dbuf, acc_sh.at[dsti.at[2 * p]], add=True)
                pltpu.make_async_copy(eh_row(j + 1), dbufb, semb).wait()

                @pl.when(p + 1 < npairs)
                def _():
                    pltpu.async_copy(eh_row(j + 2), dbuf, sema)

                pltpu.sync_copy(dbufb, acc_sh.at[dsti.at[2 * p + 1]],
                                add=True)
                return 0
            lax.fori_loop(0, npairs, pair, 0)
            return 0
        lax.fori_loop(0, nblocks, s1_block, 0)

    @pl.when(c == 0)
    def _():
        scatter_half(0)

    @pl.when(c == 1)
    def _():
        scatter_half(HALF)

    plsc.subcore_barrier()

    @pl.when(c == 0)
    def _():
        _copy_out(s, my_nchunks, acc_sh, dbuf, s0_hbm)

    @pl.when(c == 1)
    def _():
        _copy_out(s, my_nchunks, acc_sh, dbuf, s1_hbm)


def _sc_c_body(dst_hbm, cnt_hbm, cnt_sh, dsti, dbuf):
    s, row_start, row_end, nblocks, my_nchunks = _tile_ranges()

    _zero_shared(s, my_nchunks, dbuf, cnt_sh)
    plsc.subcore_barrier()
    _fill_rows(dbuf, HALF, 1.0)

    def c_block(b, _):
        jb = row_start + b * IBLK
        pltpu.sync_copy(dst_hbm.at[pl.ds(jb, IBLK)], dsti)

        def cs(jj, _):
            pltpu.sync_copy(dbuf, cnt_sh.at[dsti.at[jj]], add=True)
            return 0
        lax.fori_loop(0, jnp.minimum(row_end - jb, IBLK), cs, 0)
        return 0
    lax.fori_loop(0, nblocks, c_block, 0)
    plsc.subcore_barrier()

    _copy_out(s, my_nchunks, cnt_sh, dbuf, cnt_hbm)


def _sc_b_body(h0_hbm, h1_hbm, src_hbm, dst_hbm, o0_hbm, o1_hbm,
               acc_sh, dsti, srci, dbuf, dbufb, sema, semb):
    c = lax.axis_index("c")
    s, row_start, row_end, nblocks, my_nchunks = _tile_ranges()

    _zero_shared(s, my_nchunks, dbuf, acc_sh)
    plsc.subcore_barrier()

    def gather_half(hor_hbm):
        def s2_block(b, _):
            jb = row_start + b * IBLK
            pltpu.sync_copy(dst_hbm.at[pl.ds(jb, IBLK)], dsti)
            pltpu.sync_copy(src_hbm.at[pl.ds(jb, IBLK)], srci)
            npairs = jnp.minimum(row_end - jb, IBLK) // 2
            pltpu.async_copy(hor_hbm.at[srci.at[0]], dbuf, sema)

            def pair(p, _):
                pltpu.make_async_copy(hor_hbm.at[srci.at[2 * p]], dbuf,
                                      sema).wait()
                pltpu.async_copy(hor_hbm.at[srci.at[2 * p + 1]], dbufb,
                                 semb)
                pltpu.sync_copy(dbuf, acc_sh.at[dsti.at[2 * p]], add=True)
                pltpu.make_async_copy(hor_hbm.at[srci.at[2 * p + 1]],
                                      dbufb, semb).wait()

                @pl.when(p + 1 < npairs)
                def _():
                    pltpu.async_copy(hor_hbm.at[srci.at[2 * p + 2]], dbuf,
                                     sema)

                pltpu.sync_copy(dbufb, acc_sh.at[dsti.at[2 * p + 1]],
                                add=True)
                return 0
            lax.fori_loop(0, npairs, pair, 0)
            return 0
        lax.fori_loop(0, nblocks, s2_block, 0)

    @pl.when(c == 0)
    def _():
        gather_half(h0_hbm)

    @pl.when(c == 1)
    def _():
        gather_half(h1_hbm)

    plsc.subcore_barrier()

    @pl.when(c == 0)
    def _():
        _copy_out(s, my_nchunks, acc_sh, dbuf, o0_hbm)

    @pl.when(c == 1)
    def _():
        _copy_out(s, my_nchunks, acc_sh, dbuf, o1_hbm)


def _tc_hor_body(x_ref, s0_ref, s1_ref, cnt_ref, h0_ref, h1_ref):
    inv = 1.0 / jnp.maximum(cnt_ref[:, 0:1], 1.0)
    h0_ref[...] = x_ref[:, :HALF] * s0_ref[...] * inv
    h1_ref[...] = x_ref[:, HALF:] * s1_ref[...] * inv


def _tc_out_body(o0_ref, o1_ref, cnt_ref, w0_ref, w1_ref, b_ref, norm_ref,
                 bias_ref, out_ref):
    acc = jnp.dot(o0_ref[...], w0_ref[...],
                  preferred_element_type=jnp.float32)
    acc += jnp.dot(o1_ref[...], w1_ref[...],
                   preferred_element_type=jnp.float32)
    acc += cnt_ref[:, 0:1] * b_ref[...]
    out_ref[...] = acc * norm_ref[...] + bias_ref[...]


def kernel(x, edge_index, e_h, norm, W_inv, b_inv, bias_v):
    ei = edge_index.astype(jnp.int32)
    pad = EROWS_PAD * HALF - N_EDGES
    src = jnp.pad(ei[0], (0, pad)).reshape(EROWS_PAD, HALF)
    dst = jnp.pad(ei[1], (0, pad)).reshape(EROWS_PAD, HALF)
    mesh = plsc.VectorSubcoreMesh(core_axis_name="c", subcore_axis_name="s")
    f32 = jnp.float32

    sc_a = pl.kernel(
        _sc_a_body,
        out_type=[
            jax.ShapeDtypeStruct((N_NODES, HALF), f32),
            jax.ShapeDtypeStruct((N_NODES, HALF), f32),
        ],
        mesh=mesh,
        scratch_types=[
            pltpu.VMEM_SHARED((N_NODES, HALF), f32),
            pltpu.VMEM((IBLK, HALF), jnp.int32),
            pltpu.VMEM((HALF, HALF), f32),
            pltpu.VMEM((HALF, HALF), f32),
            pltpu.SemaphoreType.DMA,
            pltpu.SemaphoreType.DMA,
        ],
    )
    s0, s1 = sc_a(e_h, dst)

    sc_c = pl.kernel(
        _sc_c_body,
        out_type=jax.ShapeDtypeStruct((N_NODES, HALF), f32),
        mesh=mesh,
        scratch_types=[
            pltpu.VMEM_SHARED((N_NODES, HALF), f32),
            pltpu.VMEM((IBLK, HALF), jnp.int32),
            pltpu.VMEM((HALF, HALF), f32),
        ],
    )
    cnt = sc_c(dst)

    blk = 2000
    grid = N_NODES // blk
    h0, h1 = pl.pallas_call(
        _tc_hor_body,
        grid=(grid,),
        in_specs=[
            pl.BlockSpec((blk, D), lambda i: (i, 0)),
            pl.BlockSpec((blk, HALF), lambda i: (i, 0)),
            pl.BlockSpec((blk, HALF), lambda i: (i, 0)),
            pl.BlockSpec((blk, HALF), lambda i: (i, 0)),
        ],
        out_specs=[
            pl.BlockSpec((blk, HALF), lambda i: (i, 0)),
            pl.BlockSpec((blk, HALF), lambda i: (i, 0)),
        ],
        out_shape=[
            jax.ShapeDtypeStruct((N_NODES, HALF), f32),
            jax.ShapeDtypeStruct((N_NODES, HALF), f32),
        ],
    )(x, s0, s1, cnt)

    sc_b = pl.kernel(
        _sc_b_body,
        out_type=[
            jax.ShapeDtypeStruct((N_NODES, HALF), f32),
            jax.ShapeDtypeStruct((N_NODES, HALF), f32),
        ],
        mesh=mesh,
        scratch_types=[
            pltpu.VMEM_SHARED((N_NODES, HALF), f32),
            pltpu.VMEM((IBLK, HALF), jnp.int32),
            pltpu.VMEM((IBLK, HALF), jnp.int32),
            pltpu.VMEM((HALF, HALF), f32),
            pltpu.VMEM((HALF, HALF), f32),
            pltpu.SemaphoreType.DMA,
            pltpu.SemaphoreType.DMA,
        ],
    )
    o0, o1 = sc_b(h0, h1, src, dst)

    out = pl.pallas_call(
        _tc_out_body,
        grid=(grid,),
        in_specs=[
            pl.BlockSpec((blk, HALF), lambda i: (i, 0)),
            pl.BlockSpec((blk, HALF), lambda i: (i, 0)),
            pl.BlockSpec((blk, HALF), lambda i: (i, 0)),
            pl.BlockSpec((HALF, D), lambda i: (0, 0)),
            pl.BlockSpec((HALF, D), lambda i: (0, 0)),
            pl.BlockSpec((1, D), lambda i: (0, 0)),
            pl.BlockSpec((blk, 1), lambda i: (i, 0)),
            pl.BlockSpec((1, D), lambda i: (0, 0)),
        ],
        out_specs=pl.BlockSpec((blk, D), lambda i: (i, 0)),
        out_shape=jax.ShapeDtypeStruct((N_NODES, D), f32),
    )(o0, o1, cnt, W_inv[:HALF], W_inv[HALF:], b_inv.reshape(1, D),
      norm, bias_v.reshape(1, D))
    return out


# speedup vs baseline: 7.1630x; 1.0706x over previous
import jax
import jax.numpy as jnp
from jax import lax
from jax.experimental import pallas as pl
from jax.experimental.pallas import tpu as pltpu
from jax.experimental.pallas import tpu_sc as plsc

N_NODES = 10000
N_EDGES = 160000
D = 256
HALF = 128
LANES = 16
NS = 16
EROWS = N_EDGES // HALF
EROWS_PAD = 1280
EROWS_PER_TILE = 80
IBLK = 16
R = 80
NCHUNKS = N_NODES // R


def _tile_ranges():
    s = lax.axis_index("s")
    row_start = s * EROWS_PER_TILE
    row_end = jnp.minimum(row_start + EROWS_PER_TILE, EROWS)
    nblocks = (row_end - row_start + (IBLK - 1)) // IBLK
    my_nchunks = jnp.where(s < NCHUNKS - 7 * NS, 8, 7)
    return s, row_start, row_end, nblocks, my_nchunks


def _fill_rows(buf, nrows, value):
    v16 = jnp.full((LANES,), value, jnp.float32)

    def frow(r, _):
        for j in range(HALF // LANES):
            buf[r, pl.ds(j * LANES, LANES)] = v16
        return 0
    lax.fori_loop(0, nrows, frow, 0)


def _zero_shared(s, my_nchunks, dbuf, acc_sh):
    _fill_rows(dbuf, R, 0.0)

    def zchunk(i, _):
        rb = (s + NS * i) * R
        pltpu.sync_copy(dbuf.at[pl.ds(0, R)], acc_sh.at[pl.ds(rb, R)])
        return 0
    lax.fori_loop(0, my_nchunks, zchunk, 0)


def _copy_out(s, my_nchunks, acc_sh, dbuf, out_hbm):
    def ochunk(i, _):
        rb = (s + NS * i) * R
        pltpu.sync_copy(acc_sh.at[pl.ds(rb, R)], dbuf.at[pl.ds(0, R)])
        pltpu.sync_copy(dbuf.at[pl.ds(0, R)], out_hbm.at[pl.ds(rb, R)])
        return 0
    lax.fori_loop(0, my_nchunks, ochunk, 0)


def _sc_a_body(eh_hbm, dst_hbm, s0_hbm, s1_hbm,
               acc_sh, dsti, dbuf, dbufb, sema, semb):
    c = lax.axis_index("c")
    s, row_start, row_end, nblocks, my_nchunks = _tile_ranges()

    _zero_shared(s, my_nchunks, dbuf, acc_sh)
    plsc.subcore_barrier()

    def scatter_half(col0):
        def eh_row(j):
            return eh_hbm.at[pl.ds(j * HALF, HALF), pl.ds(col0, HALF)]

        def s1_block(b, _):
            jb = row_start + b * IBLK
            pltpu.sync_copy(dst_hbm.at[pl.ds(jb, IBLK)], dsti)
            npairs = jnp.minimum(row_end - jb, IBLK) // 2
            pltpu.async_copy(eh_row(jb), dbuf, sema)

            def pair(p, _):
                j = jb + 2 * p
                pltpu.make_async_copy(eh_row(j), dbuf, sema).wait()
                pltpu.async_copy(eh_row(j + 1), dbufb, semb)
                pltpu.sync_copy(dbuf, acc_sh.at[dsti.at[2 * p]], add=True)
                pltpu.make_async_copy(eh_row(j + 1), dbufb, semb).wait()

                @pl.when(p + 1 < npairs)
                def _():
                    pltpu.async_copy(eh_row(j + 2), dbuf, sema)

                pltpu.sync_copy(dbufb, acc_sh.at[dsti.at[2 * p + 1]],
                                add=True)
                return 0
            lax.fori_loop(0, npairs, pair, 0)
            return 0
        lax.fori_loop(0, nblocks, s1_block, 0)

    @pl.when(c == 0)
    def _():
        scatter_half(0)

    @pl.when(c == 1)
    def _():
        scatter_half(HALF)

    plsc.subcore_barrier()

    @pl.when(c == 0)
    def _():
        _copy_out(s, my_nchunks, acc_sh, dbuf, s0_hbm)

    @pl.when(c == 1)
    def _():
        _copy_out(s, my_nchunks, acc_sh, dbuf, s1_hbm)


def _sc_c_body(dst_hbm, cnt0_hbm, cnt1_hbm, cnt_sh, dsti, dbuf):
    c = lax.axis_index("c")
    s, _, _, _, my_nchunks = _tile_ranges()
    row_start = jnp.where(c == 0, s * 40, 640 + s * 40)
    row_end = jnp.minimum(row_start + 40, jnp.where(c == 0, 640, EROWS))
    nblocks = (row_end - row_start + (IBLK - 1)) // IBLK

    _zero_shared(s, my_nchunks, dbuf, cnt_sh)
    plsc.subcore_barrier()
    _fill_rows(dbuf, HALF, 1.0)

    def c_block(b, _):
        jb = row_start + b * IBLK
        pltpu.sync_copy(dst_hbm.at[pl.ds(jb, IBLK)], dsti)

        def cs(jj, _):
            pltpu.sync_copy(dbuf, cnt_sh.at[dsti.at[jj]], add=True)
            return 0
        lax.fori_loop(0, jnp.minimum(row_end - jb, IBLK), cs, 0)
        return 0
    lax.fori_loop(0, nblocks, c_block, 0)
    plsc.subcore_barrier()

    @pl.when(c == 0)
    def _():
        _copy_out(s, my_nchunks, cnt_sh, dbuf, cnt0_hbm)

    @pl.when(c == 1)
    def _():
        _copy_out(s, my_nchunks, cnt_sh, dbuf, cnt1_hbm)


def _sc_b_body(h0_hbm, h1_hbm, src_hbm, dst_hbm, o0_hbm, o1_hbm,
               acc_sh, dsti, srci, dbuf, dbufb, sema, semb):
    c = lax.axis_index("c")
    s, row_start, row_end, nblocks, my_nchunks = _tile_ranges()

    _zero_shared(s, my_nchunks, dbuf, acc_sh)
    plsc.subcore_barrier()

    def gather_half(hor_hbm):
        def s2_block(b, _):
            jb = row_start + b * IBLK
            pltpu.sync_copy(dst_hbm.at[pl.ds(jb, IBLK)], dsti)
            pltpu.sync_copy(src_hbm.at[pl.ds(jb, IBLK)], srci)
            npairs = jnp.minimum(row_end - jb, IBLK) // 2
            pltpu.async_copy(hor_hbm.at[srci.at[0]], dbuf, sema)

            def pair(p, _):
                pltpu.make_async_copy(hor_hbm.at[srci.at[2 * p]], dbuf,
                                      sema).wait()
                pltpu.async_copy(hor_hbm.at[srci.at[2 * p + 1]], dbufb,
                                 semb)
                pltpu.sync_copy(dbuf, acc_sh.at[dsti.at[2 * p]], add=True)
                pltpu.make_async_copy(hor_hbm.at[srci.at[2 * p + 1]],
                                      dbufb, semb).wait()

                @pl.when(p + 1 < npairs)
                def _():
                    pltpu.async_copy(hor_hbm.at[srci.at[2 * p + 2]], dbuf,
                                     sema)

                pltpu.sync_copy(dbufb, acc_sh.at[dsti.at[2 * p + 1]],
                                add=True)
                return 0
            lax.fori_loop(0, npairs, pair, 0)
            return 0
        lax.fori_loop(0, nblocks, s2_block, 0)

    @pl.when(c == 0)
    def _():
        gather_half(h0_hbm)

    @pl.when(c == 1)
    def _():
        gather_half(h1_hbm)

    plsc.subcore_barrier()

    @pl.when(c == 0)
    def _():
        _copy_out(s, my_nchunks, acc_sh, dbuf, o0_hbm)

    @pl.when(c == 1)
    def _():
        _copy_out(s, my_nchunks, acc_sh, dbuf, o1_hbm)


def _tc_hor_body(x_ref, s0_ref, s1_ref, c0_ref, c1_ref, h0_ref, h1_ref,
                 cnt_ref):
    cnt = c0_ref[:, 0:1] + c1_ref[:, 0:1]
    cnt_ref[...] = cnt
    inv = 1.0 / jnp.maximum(cnt, 1.0)
    h0_ref[...] = x_ref[:, :HALF] * s0_ref[...] * inv
    h1_ref[...] = x_ref[:, HALF:] * s1_ref[...] * inv


def _tc_out_body(o0_ref, o1_ref, cnt_ref, w0_ref, w1_ref, b_ref, norm_ref,
                 bias_ref, out_ref):
    acc = jnp.dot(o0_ref[...], w0_ref[...],
                  preferred_element_type=jnp.float32)
    acc += jnp.dot(o1_ref[...], w1_ref[...],
                   preferred_element_type=jnp.float32)
    acc += cnt_ref[...] * b_ref[...]
    out_ref[...] = acc * norm_ref[...] + bias_ref[...]


def kernel(x, edge_index, e_h, norm, W_inv, b_inv, bias_v):
    ei = edge_index.astype(jnp.int32)
    pad = EROWS_PAD * HALF - N_EDGES
    src = jnp.pad(ei[0], (0, pad)).reshape(EROWS_PAD, HALF)
    dst = jnp.pad(ei[1], (0, pad)).reshape(EROWS_PAD, HALF)
    mesh = plsc.VectorSubcoreMesh(core_axis_name="c", subcore_axis_name="s")
    f32 = jnp.float32

    sc_a = pl.kernel(
        _sc_a_body,
        out_type=[
            jax.ShapeDtypeStruct((N_NODES, HALF), f32),
            jax.ShapeDtypeStruct((N_NODES, HALF), f32),
        ],
        mesh=mesh,
        scratch_types=[
            pltpu.VMEM_SHARED((N_NODES, HALF), f32),
            pltpu.VMEM((IBLK, HALF), jnp.int32),
            pltpu.VMEM((HALF, HALF), f32),
            pltpu.VMEM((HALF, HALF), f32),
            pltpu.SemaphoreType.DMA,
            pltpu.SemaphoreType.DMA,
        ],
    )
    s0, s1 = sc_a(e_h, dst)

    sc_c = pl.kernel(
        _sc_c_body,
        out_type=[
            jax.ShapeDtypeStruct((N_NODES, HALF), f32),
            jax.ShapeDtypeStruct((N_NODES, HALF), f32),
        ],
        mesh=mesh,
        scratch_types=[
            pltpu.VMEM_SHARED((N_NODES, HALF), f32),
            pltpu.VMEM((IBLK, HALF), jnp.int32),
            pltpu.VMEM((HALF, HALF), f32),
        ],
    )
    cnt0, cnt1 = sc_c(dst)

    blk = 2000
    grid = N_NODES // blk
    h0, h1, cnt = pl.pallas_call(
        _tc_hor_body,
        grid=(grid,),
        in_specs=[
            pl.BlockSpec((blk, D), lambda i: (i, 0)),
            pl.BlockSpec((blk, HALF), lambda i: (i, 0)),
            pl.BlockSpec((blk, HALF), lambda i: (i, 0)),
            pl.BlockSpec((blk, HALF), lambda i: (i, 0)),
            pl.BlockSpec((blk, HALF), lambda i: (i, 0)),
        ],
        out_specs=[
            pl.BlockSpec((blk, HALF), lambda i: (i, 0)),
            pl.BlockSpec((blk, HALF), lambda i: (i, 0)),
            pl.BlockSpec((blk, 1), lambda i: (i, 0)),
        ],
        out_shape=[
            jax.ShapeDtypeStruct((N_NODES, HALF), f32),
            jax.ShapeDtypeStruct((N_NODES, HALF), f32),
            jax.ShapeDtypeStruct((N_NODES, 1), f32),
        ],
    )(x, s0, s1, cnt0, cnt1)

    sc_b = pl.kernel(
        _sc_b_body,
        out_type=[
            jax.ShapeDtypeStruct((N_NODES, HALF), f32),
            jax.ShapeDtypeStruct((N_NODES, HALF), f32),
        ],
        mesh=mesh,
        scratch_types=[
            pltpu.VMEM_SHARED((N_NODES, HALF), f32),
            pltpu.VMEM((IBLK, HALF), jnp.int32),
            pltpu.VMEM((IBLK, HALF), jnp.int32),
            pltpu.VMEM((HALF, HALF), f32),
            pltpu.VMEM((HALF, HALF), f32),
            pltpu.SemaphoreType.DMA,
            pltpu.SemaphoreType.DMA,
        ],
    )
    o0, o1 = sc_b(h0, h1, src, dst)

    out = pl.pallas_call(
        _tc_out_body,
        grid=(grid,),
        in_specs=[
            pl.BlockSpec((blk, HALF), lambda i: (i, 0)),
            pl.BlockSpec((blk, HALF), lambda i: (i, 0)),
            pl.BlockSpec((blk, 1), lambda i: (i, 0)),
            pl.BlockSpec((HALF, D), lambda i: (0, 0)),
            pl.BlockSpec((HALF, D), lambda i: (0, 0)),
            pl.BlockSpec((1, D), lambda i: (0, 0)),
            pl.BlockSpec((blk, 1), lambda i: (i, 0)),
            pl.BlockSpec((1, D), lambda i: (0, 0)),
        ],
        out_specs=pl.BlockSpec((blk, D), lambda i: (i, 0)),
        out_shape=jax.ShapeDtypeStruct((N_NODES, D), f32),
    )(o0, o1, cnt, W_inv[:HALF], W_inv[HALF:], b_inv.reshape(1, D),
      norm, bias_v.reshape(1, D))
    return out
